# baseline (speedup 1.0000x reference)
import functools

import jax
import jax.numpy as jnp
from jax import lax
from jax.experimental import pallas as pl
from jax.experimental.pallas import tpu as pltpu
from jax.experimental.pallas import tpu_sc as plsc

TM = 256
TN_A = 1408
TN_B = 1024
NW = 32
SC_LANES = 16


def _gemm_a_body(te_ref, xs_ref, wg_ref, wu_ref, out_ref):
    xb = xs_ref[...].astype(jnp.bfloat16)
    g = jnp.dot(xb, wg_ref[0], preferred_element_type=jnp.float32)
    u = jnp.dot(xb, wu_ref[0], preferred_element_type=jnp.float32)
    out_ref[...] = (g * jax.nn.sigmoid(g) * u).astype(jnp.bfloat16)


def _gemm_b_body(te_ref, act_ref, wd_ref, ws_ref, out_ref):
    y = jnp.dot(act_ref[...], wd_ref[0], preferred_element_type=jnp.float32)
    out_ref[...] = y * ws_ref[...]


def _grouped_ffn(xs, wg_all, wu_all, wd_all, w_slot, tile_expert):
    s_pad, h = xs.shape
    i_dim = wg_all.shape[2]
    mt = s_pad // TM

    act = pl.pallas_call(
        _gemm_a_body,
        grid_spec=pltpu.PrefetchScalarGridSpec(
            num_scalar_prefetch=1,
            grid=(i_dim // TN_A, mt),
            in_specs=[
                pl.BlockSpec((TM, h), lambda n, m, te: (m, 0)),
                pl.BlockSpec((1, h, TN_A), lambda n, m, te: (te[m], 0, n)),
                pl.BlockSpec((1, h, TN_A), lambda n, m, te: (te[m], 0, n)),
            ],
            out_specs=pl.BlockSpec((TM, TN_A), lambda n, m, te: (m, n)),
        ),
        out_shape=jax.ShapeDtypeStruct((s_pad, i_dim), jnp.bfloat16),
    )(tile_expert, xs, wg_all, wu_all)

    ys = pl.pallas_call(
        _gemm_b_body,
        grid_spec=pltpu.PrefetchScalarGridSpec(
            num_scalar_prefetch=1,
            grid=(h // TN_B, mt),
            in_specs=[
                pl.BlockSpec((TM, i_dim), lambda n, m, te: (m, 0)),
                pl.BlockSpec((1, i_dim, TN_B), lambda n, m, te: (te[m], 0, n)),
                pl.BlockSpec((TM, 1), lambda n, m, te: (m, 0)),
            ],
            out_specs=pl.BlockSpec((TM, TN_B), lambda n, m, te: (m, n)),
        ),
        out_shape=jax.ShapeDtypeStruct((s_pad, h), jnp.float32),
    )(tile_expert, act, wd_all, w_slot)
    return ys


def _sc_gather(x, row_id, s_pad):
    n, h = x.shape
    bpw = s_pad // NW
    chunk = 88
    nch = bpw // chunk
    mesh = plsc.VectorSubcoreMesh(core_axis_name="c", subcore_axis_name="s")

    @functools.partial(
        pl.kernel, mesh=mesh,
        out_type=jax.ShapeDtypeStruct((s_pad, h), jnp.float32),
        scratch_types=[
            pltpu.VMEM((bpw,), jnp.int32),
            pltpu.VMEM((chunk, h), jnp.float32),
            pltpu.SemaphoreType.DMA,
        ],
    )
    def k(x_hbm, idx_hbm, out_hbm, idx_v, rows_v, sem):
        wid = lax.axis_index("s") * 2 + lax.axis_index("c")
        base = wid * bpw
        pltpu.sync_copy(idx_hbm.at[pl.ds(base, bpw)], idx_v)

        def body(c, carry):
            pltpu.async_copy(
                x_hbm.at[idx_v.at[pl.ds(c * chunk, chunk)]], rows_v, sem
            ).wait()
            pltpu.sync_copy(rows_v, out_hbm.at[pl.ds(base + c * chunk, chunk)])
            return carry

        lax.fori_loop(0, nch, body, 0)

    return k(x, row_id)


def _sc_combine(ys, pos2, n, h):
    tpw = n // NW
    chunk = 16
    nch = tpw // chunk
    mesh = plsc.VectorSubcoreMesh(core_axis_name="c", subcore_axis_name="s")

    @functools.partial(
        pl.kernel, mesh=mesh,
        out_type=jax.ShapeDtypeStruct((n, h), jnp.float32),
        scratch_types=[
            pltpu.VMEM((2 * tpw,), jnp.int32),
            pltpu.VMEM((2 * chunk, h), jnp.float32),
            pltpu.VMEM((chunk, h), jnp.float32),
            pltpu.SemaphoreType.DMA,
        ],
    )
    def k(ys_hbm, p2_hbm, out_hbm, idx_v, rows_v, out_v, sem):
        wid = lax.axis_index("s") * 2 + lax.axis_index("c")
        tbase = wid * tpw
        pltpu.sync_copy(p2_hbm.at[pl.ds(2 * tbase, 2 * tpw)], idx_v)

        def body(c, carry):
            t0 = tbase + c * chunk
            pltpu.async_copy(
                ys_hbm.at[idx_v.at[pl.ds(c * 2 * chunk, 2 * chunk)]],
                rows_v, sem,
            ).wait()
            pltpu.sync_copy(ys_hbm.at[pl.ds(t0, chunk)], out_v)

            def tok(j, carry2):
                def vec(v, carry3):
                    sl = pl.ds(v * SC_LANES, SC_LANES)
                    out_v[j, sl] = (out_v[j, sl] + rows_v[2 * j, sl]
                                    + rows_v[2 * j + 1, sl])
                    return carry3
                return lax.fori_loop(0, h // SC_LANES, vec, carry2)

            lax.fori_loop(0, chunk, tok, 0)
            pltpu.sync_copy(out_v, out_hbm.at[pl.ds(t0, chunk)])
            return carry

        lax.fori_loop(0, nch, body, 0)

    return k(ys, pos2)


def kernel(x, Wg_s, Wu_s, Wd_s, Wg, Wu, Wd, Wr, rbias):
    n, h = x.shape
    e_routed = Wg.shape[0]
    e_all = e_routed + 1
    top_k = 2

    logits = x @ Wr + rbias
    scores, idx = lax.top_k(logits, top_k)
    w0 = jax.nn.sigmoid(scores[:, 0] - scores[:, 1])
    wn = jnp.stack([w0, 1.0 - w0], axis=1)

    eids = jnp.arange(e_routed, dtype=idx.dtype)
    mask_r = (idx[:, 0:1] == eids[None, :]) | (idx[:, 1:2] == eids[None, :])
    mask = jnp.concatenate([jnp.ones((n, 1), bool), mask_r], axis=1)
    mi = mask.astype(jnp.int32)
    ranks = jnp.cumsum(mi, axis=0) - mi
    counts = ranks[-1] + mi[-1]
    cnt_pad = ((counts + TM - 1) // TM) * TM
    starts = jnp.concatenate(
        [jnp.zeros((1,), jnp.int32), jnp.cumsum(cnt_pad)[:-1].astype(jnp.int32)])
    s_pad = n + n * top_k + e_routed * TM
    mt = s_pad // TM

    pos = starts[None, :] + ranks
    pos_flat = jnp.where(mask, pos, s_pad).reshape(-1)
    tok = jnp.broadcast_to(jnp.arange(n, dtype=jnp.int32)[:, None],
                           (n, e_all)).reshape(-1)
    row_id = jnp.zeros((s_pad + 1,), jnp.int32).at[pos_flat].set(
        tok, mode="drop")[:s_pad]
    w_routed = (wn[:, 0:1] * (idx[:, 0:1] == eids[None, :])
                + wn[:, 1:2] * (idx[:, 1:2] == eids[None, :]))
    w_flat = jnp.concatenate([jnp.ones((n, 1), x.dtype), w_routed],
                             axis=1).reshape(-1)
    w_slot = jnp.zeros((s_pad + 1,), x.dtype).at[pos_flat].set(
        w_flat, mode="drop")[:s_pad, None]

    ends = (starts + cnt_pad).astype(jnp.int32)
    tile_off = jnp.arange(mt, dtype=jnp.int32) * TM
    tile_expert = jnp.minimum(
        jnp.searchsorted(ends, tile_off, side="right"), e_all - 1
    ).astype(jnp.int32)

    wg_all = jnp.concatenate([Wg_s[None], Wg], 0).astype(jnp.bfloat16)
    wu_all = jnp.concatenate([Wu_s[None], Wu], 0).astype(jnp.bfloat16)
    wd_all = jnp.concatenate([Wd_s[None], Wd], 0).astype(jnp.bfloat16)

    xs = _sc_gather(x, row_id, s_pad)
    ys = _grouped_ffn(xs, wg_all, wu_all, wd_all, w_slot, tile_expert)

    pos2 = jnp.take_along_axis(pos[:, 1:], idx, axis=1).reshape(-1)
    return _sc_combine(ys, pos2.astype(jnp.int32), n, h)


# speedup vs baseline: 1.0193x; 1.0193x over previous
import functools

import jax
import jax.numpy as jnp
from jax import lax
from jax.experimental import pallas as pl
from jax.experimental.pallas import tpu as pltpu
from jax.experimental.pallas import tpu_sc as plsc

TM = 256
TN_A = 1408
TN_B = 1024
NW = 32
SC_LANES = 16


def _gemm_a_body(te_ref, xs_ref, wg_ref, wu_ref, out_ref):
    xb = xs_ref[...].astype(jnp.bfloat16)
    g = jnp.dot(xb, wg_ref[0], preferred_element_type=jnp.float32)
    u = jnp.dot(xb, wu_ref[0], preferred_element_type=jnp.float32)
    out_ref[...] = (g * jax.nn.sigmoid(g) * u).astype(jnp.bfloat16)


def _gemm_b_body(te_ref, act_ref, wd_ref, ws_ref, out_ref):
    y = jnp.dot(act_ref[...], wd_ref[0], preferred_element_type=jnp.float32)
    out_ref[...] = y * ws_ref[...]


def _grouped_ffn(xs, wg_all, wu_all, wd_all, w_slot, tile_expert):
    s_pad, h = xs.shape
    i_dim = wg_all.shape[2]
    mt = s_pad // TM

    act = pl.pallas_call(
        _gemm_a_body,
        grid_spec=pltpu.PrefetchScalarGridSpec(
            num_scalar_prefetch=1,
            grid=(i_dim // TN_A, mt),
            in_specs=[
                pl.BlockSpec((TM, h), lambda n, m, te: (m, 0)),
                pl.BlockSpec((1, h, TN_A), lambda n, m, te: (te[m], 0, n)),
                pl.BlockSpec((1, h, TN_A), lambda n, m, te: (te[m], 0, n)),
            ],
            out_specs=pl.BlockSpec((TM, TN_A), lambda n, m, te: (m, n)),
        ),
        out_shape=jax.ShapeDtypeStruct((s_pad, i_dim), jnp.bfloat16),
    )(tile_expert, xs, wg_all, wu_all)

    ys = pl.pallas_call(
        _gemm_b_body,
        grid_spec=pltpu.PrefetchScalarGridSpec(
            num_scalar_prefetch=1,
            grid=(h // TN_B, mt),
            in_specs=[
                pl.BlockSpec((TM, i_dim), lambda n, m, te: (m, 0)),
                pl.BlockSpec((1, i_dim, TN_B), lambda n, m, te: (te[m], 0, n)),
                pl.BlockSpec((TM, 1), lambda n, m, te: (m, 0)),
            ],
            out_specs=pl.BlockSpec((TM, TN_B), lambda n, m, te: (m, n)),
        ),
        out_shape=jax.ShapeDtypeStruct((s_pad, h), jnp.float32),
    )(tile_expert, act, wd_all, w_slot)
    return ys


def _sc_gather(x, row_id, s_pad):
    n, h = x.shape
    bpw = s_pad // NW
    chunk = 40
    nch = bpw // chunk
    mesh = plsc.VectorSubcoreMesh(core_axis_name="c", subcore_axis_name="s")

    @functools.partial(
        pl.kernel, mesh=mesh,
        out_type=jax.ShapeDtypeStruct((s_pad, h), jnp.float32),
        scratch_types=[
            pltpu.VMEM((bpw,), jnp.int32),
            pltpu.VMEM((chunk, h), jnp.float32),
            pltpu.VMEM((chunk, h), jnp.float32),
            pltpu.SemaphoreType.DMA,
            pltpu.SemaphoreType.DMA,
            pltpu.SemaphoreType.DMA,
            pltpu.SemaphoreType.DMA,
        ],
    )
    def k(x_hbm, idx_hbm, out_hbm, idx_v, rows0, rows1, gs0, gs1, ws0, ws1):
        wid = lax.axis_index("s") * 2 + lax.axis_index("c")
        base = wid * bpw
        pltpu.sync_copy(idx_hbm.at[pl.ds(base, bpw)], idx_v)
        rows = (rows0, rows1)
        gsem = (gs0, gs1)
        wsem = (ws0, ws1)
        gh = [None] * nch
        wh = [None] * nch

        def start_gather(c):
            gh[c] = pltpu.async_copy(
                x_hbm.at[idx_v.at[pl.ds(c * chunk, chunk)]],
                rows[c % 2], gsem[c % 2])

        start_gather(0)
        for c in range(nch):
            if c + 1 < nch:
                if c >= 1:
                    wh[c - 1].wait()
                start_gather(c + 1)
            gh[c].wait()
            wh[c] = pltpu.async_copy(
                rows[c % 2], out_hbm.at[pl.ds(base + c * chunk, chunk)],
                wsem[c % 2])
        wh[nch - 2].wait()
        wh[nch - 1].wait()

    return k(x, row_id)


def _sc_combine(ys, pos2, n, h):
    tpw = n // NW
    chunk = 16
    nch = tpw // chunk
    mesh = plsc.VectorSubcoreMesh(core_axis_name="c", subcore_axis_name="s")

    @functools.partial(
        pl.kernel, mesh=mesh,
        out_type=jax.ShapeDtypeStruct((n, h), jnp.float32),
        scratch_types=[
            pltpu.VMEM((2 * tpw,), jnp.int32),
            pltpu.VMEM((2 * chunk, h), jnp.float32),
            pltpu.VMEM((2 * chunk, h), jnp.float32),
            pltpu.VMEM((chunk, h), jnp.float32),
            pltpu.VMEM((chunk, h), jnp.float32),
            pltpu.SemaphoreType.DMA,
            pltpu.SemaphoreType.DMA,
            pltpu.SemaphoreType.DMA,
            pltpu.SemaphoreType.DMA,
            pltpu.SemaphoreType.DMA,
            pltpu.SemaphoreType.DMA,
        ],
    )
    def k(ys_hbm, p2_hbm, out_hbm, idx_v, rows0, rows1, out0, out1,
          gr0, gr1, ln0, ln1, wb0, wb1):
        wid = lax.axis_index("s") * 2 + lax.axis_index("c")
        tbase = wid * tpw
        pltpu.sync_copy(p2_hbm.at[pl.ds(2 * tbase, 2 * tpw)], idx_v)
        rows = (rows0, rows1)
        outs = (out0, out1)
        grs = (gr0, gr1)
        lns = (ln0, ln1)
        wbs = (wb0, wb1)
        gh = [None] * nch
        lh = [None] * nch
        wh = [None] * nch

        def start(c):
            gh[c] = pltpu.async_copy(
                ys_hbm.at[idx_v.at[pl.ds(c * 2 * chunk, 2 * chunk)]],
                rows[c % 2], grs[c % 2])
            lh[c] = pltpu.async_copy(
                ys_hbm.at[pl.ds(tbase + c * chunk, chunk)],
                outs[c % 2], lns[c % 2])

        start(0)
        for c in range(nch):
            if c + 1 < nch:
                if c >= 1:
                    wh[c - 1].wait()
                start(c + 1)
            gh[c].wait()
            lh[c].wait()
            rows_v = rows[c % 2]
            out_v = outs[c % 2]

            def tok(j, carry2):
                def vec(v, carry3):
                    sl = pl.ds(v * SC_LANES, SC_LANES)
                    out_v[j, sl] = (out_v[j, sl] + rows_v[2 * j, sl]
                                    + rows_v[2 * j + 1, sl])
                    return carry3
                return lax.fori_loop(0, h // SC_LANES, vec, carry2)

            lax.fori_loop(0, chunk, tok, 0)
            wh[c] = pltpu.async_copy(
                out_v, out_hbm.at[pl.ds(tbase + c * chunk, chunk)],
                wbs[c % 2])
        wh[nch - 2].wait()
        wh[nch - 1].wait()

    return k(ys, pos2)


def kernel(x, Wg_s, Wu_s, Wd_s, Wg, Wu, Wd, Wr, rbias):
    n, h = x.shape
    e_routed = Wg.shape[0]
    e_all = e_routed + 1
    top_k = 2

    logits = x @ Wr + rbias
    scores, idx = lax.top_k(logits, top_k)
    w0 = jax.nn.sigmoid(scores[:, 0] - scores[:, 1])
    wn = jnp.stack([w0, 1.0 - w0], axis=1)

    eids = jnp.arange(e_routed, dtype=idx.dtype)
    mask_r = (idx[:, 0:1] == eids[None, :]) | (idx[:, 1:2] == eids[None, :])
    mask = jnp.concatenate([jnp.ones((n, 1), bool), mask_r], axis=1)
    mi = mask.astype(jnp.int32)
    ranks = jnp.cumsum(mi, axis=0) - mi
    counts = ranks[-1] + mi[-1]
    cnt_pad = ((counts + TM - 1) // TM) * TM
    starts = jnp.concatenate(
        [jnp.zeros((1,), jnp.int32), jnp.cumsum(cnt_pad)[:-1].astype(jnp.int32)])
    s_pad = n + n * top_k + e_routed * TM
    mt = s_pad // TM

    pos = starts[None, :] + ranks
    pos_flat = jnp.where(mask, pos, s_pad).reshape(-1)
    tok = jnp.broadcast_to(jnp.arange(n, dtype=jnp.int32)[:, None],
                           (n, e_all)).reshape(-1)
    row_id = jnp.zeros((s_pad + 1,), jnp.int32).at[pos_flat].set(
        tok, mode="drop")[:s_pad]
    w_routed = (wn[:, 0:1] * (idx[:, 0:1] == eids[None, :])
                + wn[:, 1:2] * (idx[:, 1:2] == eids[None, :]))
    w_flat = jnp.concatenate([jnp.ones((n, 1), x.dtype), w_routed],
                             axis=1).reshape(-1)
    w_slot = jnp.zeros((s_pad + 1,), x.dtype).at[pos_flat].set(
        w_flat, mode="drop")[:s_pad, None]

    ends = (starts + cnt_pad).astype(jnp.int32)
    tile_off = jnp.arange(mt, dtype=jnp.int32) * TM
    tile_expert = jnp.minimum(
        jnp.searchsorted(ends, tile_off, side="right"), e_all - 1
    ).astype(jnp.int32)

    wg_all = jnp.concatenate([Wg_s[None], Wg], 0).astype(jnp.bfloat16)
    wu_all = jnp.concatenate([Wu_s[None], Wu], 0).astype(jnp.bfloat16)
    wd_all = jnp.concatenate([Wd_s[None], Wd], 0).astype(jnp.bfloat16)

    xs = _sc_gather(x, row_id, s_pad)
    ys = _grouped_ffn(xs, wg_all, wu_all, wd_all, w_slot, tile_expert)

    pos2 = jnp.take_along_axis(pos[:, 1:], idx, axis=1).reshape(-1)
    return _sc_combine(ys, pos2.astype(jnp.int32), n, h)
